# baseline (speedup 1.0000x reference)
import functools

import jax
import jax.numpy as jnp
from jax import lax
from jax.experimental import pallas as pl
from jax.experimental.pallas import tpu as pltpu
from jax.experimental.pallas import tpu_sc as plsc

_B, _A, _R, _L = 4, 2, 64, 65536
_BA = _B * _A
_NC, _NS, _LANES = 2, 16, 16
_NW = _NC * _NS
_WPR = _NW // _BA

_L_TC = 40960
_L_SC = _L - _L_TC
_LW = _L_SC // _WPR
_C = 512
_NCHUNK = _LW // _C

_CT = 2048


def _sc_call(mixed, ref, wpad):
    mesh = plsc.VectorSubcoreMesh(core_axis_name="c", subcore_axis_name="s")

    @functools.partial(
        pl.kernel,
        mesh=mesh,
        out_type=[
            jax.ShapeDtypeStruct((_BA, _L_SC), jnp.float32),
            jax.ShapeDtypeStruct((_BA, _L_SC), jnp.int32),
        ],
        scratch_types=[
            pltpu.VMEM((2, _R, _C), jnp.float32),
            pltpu.VMEM((_LW,), jnp.float32),
            pltpu.VMEM((_LW,), jnp.float32),
            pltpu.VMEM((_LW,), jnp.int32),
            pltpu.VMEM((_LANES,), jnp.float32),
            pltpu.SemaphoreType.DMA,
            pltpu.SemaphoreType.DMA,
        ],
    )
    def body(mixed_hbm, ref_hbm, w_hbm, pooled_hbm, idx_hbm,
             ref_v, m_v, p_v, i_v, w_v, sem0, sem1):
        wid = lax.axis_index("s") * _NC + lax.axis_index("c")
        ba = wid // _WPR
        b = ba // _A
        span0 = (wid % _WPR) * _LW
        l_base = _L_TC + span0
        sems = (sem0, sem1)

        def start_in(ci, par):
            l0 = l_base + ci * _C
            pltpu.async_copy(
                ref_hbm.at[ba, :, pl.ds(l0, _C)], ref_v.at[par], sems[par])

        def wait_in(par):
            pltpu.make_async_copy(
                ref_hbm.at[ba, :, pl.ds(l_base, _C)], ref_v.at[par],
                sems[par]).wait()

        pltpu.sync_copy(w_hbm, w_v)
        pltpu.sync_copy(mixed_hbm.at[b, pl.ds(l_base, _LW)], m_v)
        wvec = w_v[...]
        w0 = wvec[0]
        w1 = wvec[1]

        start_in(0, 0)
        start_in(1, 1)

        def compute(ci, par):
            off0 = ci * _C

            def j_body(j, _):
                off = off0 + j * _LANES
                m = m_v[pl.ds(off, _LANES)]

                def r_body(r, carry):
                    mx1, mx2, ix = carry
                    v = ref_v[par, r, pl.ds(j * _LANES, _LANES)] * m
                    gt = v > mx1
                    mx2n = jnp.maximum(mx2, jnp.minimum(v, mx1))
                    ixn = jnp.where(gt, jnp.full((_LANES,), 0, jnp.int32) + r, ix)
                    mx1n = jnp.maximum(mx1, v)
                    return mx1n, mx2n, ixn

                neg = jnp.full((_LANES,), -jnp.inf, jnp.float32)
                mx1, mx2, ix = lax.fori_loop(
                    0, _R, r_body,
                    (neg, neg, jnp.zeros((_LANES,), jnp.int32)),
                    unroll=8)
                p_v[pl.ds(off, _LANES)] = mx1 * w0 + mx2 * w1
                i_v[pl.ds(off, _LANES)] = ix
                return 0

            lax.fori_loop(0, _C // _LANES, j_body, 0)

        def chunk_pair(cp, _):
            ci0 = cp * 2
            wait_in(0)
            compute(ci0, 0)

            @pl.when(ci0 + 2 < _NCHUNK)
            def _():
                start_in(ci0 + 2, 0)

            wait_in(1)
            compute(ci0 + 1, 1)

            @pl.when(ci0 + 3 < _NCHUNK)
            def _():
                start_in(ci0 + 3, 1)

            return 0

        lax.fori_loop(0, _NCHUNK // 2, chunk_pair, 0)

        pltpu.sync_copy(p_v, pooled_hbm.at[ba, pl.ds(span0, _LW)])
        pltpu.sync_copy(i_v, idx_hbm.at[ba, pl.ds(span0, _LW)])

    return body(mixed, ref, wpad)


def _tc_body(m_ref, r_ref, w_ref, p_ref, i_ref):
    blk = r_ref[0]
    m = m_ref[0]
    multi = blk * m
    mx1 = jnp.max(multi, axis=0, keepdims=True)
    iota = lax.broadcasted_iota(jnp.int32, (_R, _CT), 0)
    ix = jnp.min(jnp.where(multi == mx1, iota, _R), axis=0, keepdims=True)
    masked = jnp.where(iota == ix, -jnp.inf, multi)
    mx2 = jnp.max(masked, axis=0, keepdims=True)
    w0 = w_ref[0, 0]
    w1 = w_ref[0, 1]
    p_ref[0] = mx1 * w0 + mx2 * w1
    i_ref[0] = ix


def _tc_call(mixed3, ref, w2d):
    grid = (_BA, _L_TC // _CT)
    return pl.pallas_call(
        _tc_body,
        grid=grid,
        in_specs=[
            pl.BlockSpec((1, 1, _CT), lambda ba, j: (ba // _A, 0, j)),
            pl.BlockSpec((1, _R, _CT), lambda ba, j: (ba, 0, j)),
            pl.BlockSpec((8, 128), lambda ba, j: (0, 0)),
        ],
        out_specs=[
            pl.BlockSpec((1, 1, _CT), lambda ba, j: (ba, 0, j)),
            pl.BlockSpec((1, 1, _CT), lambda ba, j: (ba, 0, j)),
        ],
        out_shape=[
            jax.ShapeDtypeStruct((_BA, 1, _L_TC), jnp.float32),
            jax.ShapeDtypeStruct((_BA, 1, _L_TC), jnp.int32),
        ],
    )(mixed3, ref, w2d)


def kernel(mixed_vcf, ref_panel, weights):
    ref = ref_panel.reshape(_BA, _R, _L)
    k = weights.shape[0]
    wpad = jnp.pad(weights.reshape(-1), (0, _LANES - k))
    w2d = jnp.broadcast_to(wpad[None, :], (8, 128)[:1] + (16,))
    w2d = jnp.pad(w2d, ((0, 7), (0, 112)))
    mixed3 = mixed_vcf.reshape(_B, 1, _L)

    p_tc, i_tc = _tc_call(mixed3, ref, w2d)
    p_sc, i_sc = _sc_call(mixed_vcf, ref, wpad)

    pooled = jnp.concatenate([p_tc.reshape(_BA, _L_TC), p_sc], axis=1)
    idx = jnp.concatenate([i_tc.reshape(_BA, _L_TC), i_sc], axis=1)
    return pooled.reshape(_B, _A, _L), idx.reshape(_B, _A, _L)


# speedup vs baseline: 1.3140x; 1.3140x over previous
import functools

import jax
import jax.numpy as jnp
from jax import lax
from jax.experimental import pallas as pl
from jax.experimental.pallas import tpu as pltpu
from jax.experimental.pallas import tpu_sc as plsc

_B, _A, _R, _L = 4, 2, 64, 65536
_BA = _B * _A
_NC, _NS, _LANES = 2, 16, 16
_NW = _NC * _NS
_WPR = _NW // _BA

_L_TC = 40960
_L_SC = _L - _L_TC
_LW = _L_SC // _WPR
_C = 512
_NCHUNK = _LW // _C

_CT = 4096


def _sc_call(mixed, ref, wpad):
    mesh = plsc.VectorSubcoreMesh(core_axis_name="c", subcore_axis_name="s")

    @functools.partial(
        pl.kernel,
        mesh=mesh,
        out_type=[
            jax.ShapeDtypeStruct((_BA, _L_SC), jnp.float32),
            jax.ShapeDtypeStruct((_BA, _L_SC), jnp.int32),
        ],
        scratch_types=[
            pltpu.VMEM((2, _R, _C), jnp.float32),
            pltpu.VMEM((_LW,), jnp.float32),
            pltpu.VMEM((_LW,), jnp.float32),
            pltpu.VMEM((_LW,), jnp.int32),
            pltpu.VMEM((_LANES,), jnp.float32),
            pltpu.SemaphoreType.DMA,
            pltpu.SemaphoreType.DMA,
        ],
    )
    def body(mixed_hbm, ref_hbm, w_hbm, pooled_hbm, idx_hbm,
             ref_v, m_v, p_v, i_v, w_v, sem0, sem1):
        wid = lax.axis_index("s") * _NC + lax.axis_index("c")
        ba = wid // _WPR
        b = ba // _A
        span0 = (wid % _WPR) * _LW
        l_base = _L_TC + span0
        sems = (sem0, sem1)

        def start_in(ci, par):
            l0 = l_base + ci * _C
            pltpu.async_copy(
                ref_hbm.at[ba, :, pl.ds(l0, _C)], ref_v.at[par], sems[par])

        def wait_in(par):
            pltpu.make_async_copy(
                ref_hbm.at[ba, :, pl.ds(l_base, _C)], ref_v.at[par],
                sems[par]).wait()

        pltpu.sync_copy(w_hbm, w_v)
        pltpu.sync_copy(mixed_hbm.at[b, pl.ds(l_base, _LW)], m_v)
        wvec = w_v[...]
        w0 = wvec[0]
        w1 = wvec[1]

        start_in(0, 0)
        start_in(1, 1)

        def compute(ci, par):
            off0 = ci * _C

            def j_body(j, _):
                off = off0 + j * _LANES
                m = m_v[pl.ds(off, _LANES)]

                def r_body(r, carry):
                    mx1, mx2, ix = carry
                    v = ref_v[par, r, pl.ds(j * _LANES, _LANES)] * m
                    gt = v > mx1
                    mx2n = jnp.maximum(mx2, jnp.minimum(v, mx1))
                    ixn = jnp.where(gt, jnp.full((_LANES,), 0, jnp.int32) + r, ix)
                    mx1n = jnp.maximum(mx1, v)
                    return mx1n, mx2n, ixn

                neg = jnp.full((_LANES,), -jnp.inf, jnp.float32)
                mx1, mx2, ix = lax.fori_loop(
                    0, _R, r_body,
                    (neg, neg, jnp.zeros((_LANES,), jnp.int32)),
                    unroll=8)
                p_v[pl.ds(off, _LANES)] = mx1 * w0 + mx2 * w1
                i_v[pl.ds(off, _LANES)] = ix
                return 0

            lax.fori_loop(0, _C // _LANES, j_body, 0)

        def chunk_pair(cp, _):
            ci0 = cp * 2
            wait_in(0)
            compute(ci0, 0)

            @pl.when(ci0 + 2 < _NCHUNK)
            def _():
                start_in(ci0 + 2, 0)

            wait_in(1)
            compute(ci0 + 1, 1)

            @pl.when(ci0 + 3 < _NCHUNK)
            def _():
                start_in(ci0 + 3, 1)

            return 0

        lax.fori_loop(0, _NCHUNK // 2, chunk_pair, 0)

        pltpu.sync_copy(p_v, pooled_hbm.at[ba, pl.ds(span0, _LW)])
        pltpu.sync_copy(i_v, idx_hbm.at[ba, pl.ds(span0, _LW)])

    return body(mixed, ref, wpad)


def _tc_body(m_ref, r_ref, w_ref, p_ref, i_ref):
    blk = r_ref[0]
    m = m_ref[0]
    multi = blk * m
    mx1 = jnp.max(multi, axis=0, keepdims=True)
    iota = lax.broadcasted_iota(jnp.int32, (_R, _CT), 0)
    ix = jnp.min(jnp.where(multi == mx1, iota, _R), axis=0, keepdims=True)
    masked = jnp.where(iota == ix, -jnp.inf, multi)
    mx2 = jnp.max(masked, axis=0, keepdims=True)
    w0 = w_ref[0, 0]
    w1 = w_ref[0, 1]
    p_ref[0] = mx1 * w0 + mx2 * w1
    i_ref[0] = ix


def _tc_call(mixed3, ref, w2d):
    grid = (_BA, _L_TC // _CT)
    return pl.pallas_call(
        _tc_body,
        grid=grid,
        in_specs=[
            pl.BlockSpec((1, 1, _CT), lambda ba, j: (ba // _A, 0, j)),
            pl.BlockSpec((1, _R, _CT), lambda ba, j: (ba, 0, j)),
            pl.BlockSpec((8, 128), lambda ba, j: (0, 0)),
        ],
        out_specs=[
            pl.BlockSpec((1, 1, _CT), lambda ba, j: (ba, 0, j)),
            pl.BlockSpec((1, 1, _CT), lambda ba, j: (ba, 0, j)),
        ],
        out_shape=[
            jax.ShapeDtypeStruct((_BA, 1, _L_TC), jnp.float32),
            jax.ShapeDtypeStruct((_BA, 1, _L_TC), jnp.int32),
        ],
    )(mixed3, ref, w2d)


def kernel(mixed_vcf, ref_panel, weights):
    ref = ref_panel.reshape(_BA, _R, _L)
    k = weights.shape[0]
    wpad = jnp.pad(weights.reshape(-1), (0, _LANES - k))
    w2d = jnp.broadcast_to(wpad[None, :], (8, 128)[:1] + (16,))
    w2d = jnp.pad(w2d, ((0, 7), (0, 112)))
    mixed3 = mixed_vcf.reshape(_B, 1, _L)

    p_tc, i_tc = _tc_call(mixed3, ref, w2d)
    p_sc, i_sc = _sc_call(mixed_vcf, ref, wpad)

    pooled = jnp.concatenate([p_tc.reshape(_BA, _L_TC), p_sc], axis=1)
    idx = jnp.concatenate([i_tc.reshape(_BA, _L_TC), i_sc], axis=1)
    return pooled.reshape(_B, _A, _L), idx.reshape(_B, _A, _L)


# speedup vs baseline: 1.8813x; 1.4317x over previous
import functools

import jax
import jax.numpy as jnp
from jax import lax
from jax.experimental import pallas as pl
from jax.experimental.pallas import tpu as pltpu
from jax.experimental.pallas import tpu_sc as plsc

_B, _A, _R, _L = 4, 2, 64, 65536
_BA = _B * _A
_NC, _NS, _LANES = 2, 16, 16
_NW = _NC * _NS

_ROWS_TC = 4
_ROWS_SC = _BA - _ROWS_TC
_WPR = _NW // _ROWS_SC
_LW = _L // _WPR
_C = 512
_NCHUNK = _LW // _C

_CT = 4096


def _sc_call(mixed, ref, w1d):
    mesh = plsc.VectorSubcoreMesh(core_axis_name="c", subcore_axis_name="s")

    @functools.partial(
        pl.kernel,
        mesh=mesh,
        out_type=[
            jax.ShapeDtypeStruct((_ROWS_SC, _L), jnp.float32),
            jax.ShapeDtypeStruct((_ROWS_SC, _L), jnp.int32),
        ],
        scratch_types=[
            pltpu.VMEM((2, _R, _C), jnp.float32),
            pltpu.VMEM((_LW,), jnp.float32),
            pltpu.VMEM((_LW,), jnp.float32),
            pltpu.VMEM((_LW,), jnp.int32),
            pltpu.VMEM((_LANES,), jnp.float32),
            pltpu.SemaphoreType.DMA,
            pltpu.SemaphoreType.DMA,
        ],
    )
    def body(mixed_hbm, ref_hbm, w_hbm, pooled_hbm, idx_hbm,
             ref_v, m_v, p_v, i_v, w_v, sem0, sem1):
        wid = lax.axis_index("s") * _NC + lax.axis_index("c")
        row = wid // _WPR
        ba = _ROWS_TC + row
        b = ba // _A
        l_base = (wid % _WPR) * _LW
        sems = (sem0, sem1)

        def start_in(ci, par):
            l0 = l_base + ci * _C
            pltpu.async_copy(
                ref_hbm.at[ba, :, pl.ds(l0, _C)], ref_v.at[par], sems[par])

        def wait_in(par):
            pltpu.make_async_copy(
                ref_hbm.at[ba, :, pl.ds(l_base, _C)], ref_v.at[par],
                sems[par]).wait()

        pltpu.sync_copy(w_hbm.at[0, pl.ds(0, _LANES)], w_v)
        pltpu.sync_copy(mixed_hbm.at[b, pl.ds(l_base, _LW)], m_v)
        wvec = w_v[...]
        w0 = wvec[0]
        w1 = wvec[1]

        start_in(0, 0)
        start_in(1, 1)

        def compute(ci, par):
            off0 = ci * _C

            def j_body(j, _):
                off = off0 + j * _LANES
                m = m_v[pl.ds(off, _LANES)]

                def r_body(r, carry):
                    mx1, mx2, ix = carry
                    v = ref_v[par, r, pl.ds(j * _LANES, _LANES)] * m
                    gt = v > mx1
                    mx2n = jnp.maximum(mx2, jnp.minimum(v, mx1))
                    ixn = jnp.where(gt, jnp.full((_LANES,), 0, jnp.int32) + r, ix)
                    mx1n = jnp.maximum(mx1, v)
                    return mx1n, mx2n, ixn

                neg = jnp.full((_LANES,), -jnp.inf, jnp.float32)
                mx1, mx2, ix = lax.fori_loop(
                    0, _R, r_body,
                    (neg, neg, jnp.zeros((_LANES,), jnp.int32)),
                    unroll=8)
                p_v[pl.ds(off, _LANES)] = mx1 * w0 + mx2 * w1
                i_v[pl.ds(off, _LANES)] = ix
                return 0

            lax.fori_loop(0, _C // _LANES, j_body, 0)

        def chunk_pair(cp, _):
            ci0 = cp * 2
            wait_in(0)
            compute(ci0, 0)

            @pl.when(ci0 + 2 < _NCHUNK)
            def _():
                start_in(ci0 + 2, 0)

            wait_in(1)
            compute(ci0 + 1, 1)

            @pl.when(ci0 + 3 < _NCHUNK)
            def _():
                start_in(ci0 + 3, 1)

            return 0

        lax.fori_loop(0, _NCHUNK // 2, chunk_pair, 0)

        pltpu.sync_copy(p_v, pooled_hbm.at[row, pl.ds(l_base, _LW)])
        pltpu.sync_copy(i_v, idx_hbm.at[row, pl.ds(l_base, _LW)])

    return body(mixed, ref, w1d)


def _tc_body(m_ref, r_ref, w_ref, p_ref, i_ref):
    blk = r_ref[...]
    m01 = m_ref[...]
    m2 = jnp.concatenate(
        [m01[0:1], m01[0:1], m01[1:2], m01[1:2]], axis=0)
    multi = blk * m2[:, None, :]
    mx1 = jnp.max(multi, axis=1, keepdims=True)
    iota = lax.broadcasted_iota(jnp.int32, (_ROWS_TC, _R, _CT), 1)
    ix = jnp.min(jnp.where(multi == mx1, iota, _R), axis=1, keepdims=True)
    masked = jnp.where(iota == ix, -jnp.inf, multi)
    mx2 = jnp.max(masked, axis=1, keepdims=True)
    w0 = w_ref[0, 0]
    w1 = w_ref[0, 1]
    p_ref[...] = (mx1 * w0 + mx2 * w1)[:, 0, :]
    i_ref[...] = ix[:, 0, :]


def _tc_call(mixed, ref, w1d):
    grid = (_L // _CT,)
    return pl.pallas_call(
        _tc_body,
        grid=grid,
        in_specs=[
            pl.BlockSpec((_B, _CT), lambda j: (0, j)),
            pl.BlockSpec((_ROWS_TC, _R, _CT), lambda j: (0, 0, j)),
            pl.BlockSpec((1, 128), lambda j: (0, 0)),
        ],
        out_specs=[
            pl.BlockSpec((_ROWS_TC, _CT), lambda j: (0, j)),
            pl.BlockSpec((_ROWS_TC, _CT), lambda j: (0, j)),
        ],
        out_shape=[
            jax.ShapeDtypeStruct((_ROWS_TC, _L), jnp.float32),
            jax.ShapeDtypeStruct((_ROWS_TC, _L), jnp.int32),
        ],
    )(mixed, ref, w1d)


def kernel(mixed_vcf, ref_panel, weights):
    ref = ref_panel.reshape(_BA, _R, _L)
    k = weights.shape[0]
    w1d = jnp.pad(weights.reshape(1, k), ((0, 0), (0, 128 - k)))

    p_tc, i_tc = _tc_call(mixed_vcf, ref, w1d)
    p_sc, i_sc = _sc_call(mixed_vcf, ref, w1d)

    pooled = jnp.concatenate([p_tc, p_sc], axis=0)
    idx = jnp.concatenate([i_tc, i_sc], axis=0)
    return pooled.reshape(_B, _A, _L), idx.reshape(_B, _A, _L)


# speedup vs baseline: 1.8967x; 1.0082x over previous
import functools

import jax
import jax.numpy as jnp
from jax import lax
from jax.experimental import pallas as pl
from jax.experimental.pallas import tpu as pltpu
from jax.experimental.pallas import tpu_sc as plsc

_B, _A, _R, _L = 4, 2, 64, 65536
_BA = _B * _A
_NC, _NS, _LANES = 2, 16, 16
_NW = _NC * _NS

_ROWS_TC = 4
_ROWS_SC = _BA - _ROWS_TC
_WPR = _NW // _ROWS_SC
_LW = _L // _WPR
_C = 512
_NCHUNK = _LW // _C

_CT = 8192


def _sc_call(mixed, ref, w1d):
    mesh = plsc.VectorSubcoreMesh(core_axis_name="c", subcore_axis_name="s")

    @functools.partial(
        pl.kernel,
        mesh=mesh,
        out_type=[
            jax.ShapeDtypeStruct((_ROWS_SC, _L), jnp.float32),
            jax.ShapeDtypeStruct((_ROWS_SC, _L), jnp.int32),
        ],
        scratch_types=[
            pltpu.VMEM((2, _R, _C), jnp.float32),
            pltpu.VMEM((_LW,), jnp.float32),
            pltpu.VMEM((_LW,), jnp.float32),
            pltpu.VMEM((_LW,), jnp.int32),
            pltpu.VMEM((_LANES,), jnp.float32),
            pltpu.SemaphoreType.DMA,
            pltpu.SemaphoreType.DMA,
        ],
    )
    def body(mixed_hbm, ref_hbm, w_hbm, pooled_hbm, idx_hbm,
             ref_v, m_v, p_v, i_v, w_v, sem0, sem1):
        wid = lax.axis_index("s") * _NC + lax.axis_index("c")
        row = wid // _WPR
        ba = _ROWS_TC + row
        b = ba // _A
        l_base = (wid % _WPR) * _LW
        sems = (sem0, sem1)

        def start_in(ci, par):
            l0 = l_base + ci * _C
            pltpu.async_copy(
                ref_hbm.at[ba, :, pl.ds(l0, _C)], ref_v.at[par], sems[par])

        def wait_in(par):
            pltpu.make_async_copy(
                ref_hbm.at[ba, :, pl.ds(l_base, _C)], ref_v.at[par],
                sems[par]).wait()

        pltpu.sync_copy(w_hbm.at[0, pl.ds(0, _LANES)], w_v)
        pltpu.sync_copy(mixed_hbm.at[b, pl.ds(l_base, _LW)], m_v)
        wvec = w_v[...]
        w0 = wvec[0]
        w1 = wvec[1]

        start_in(0, 0)
        start_in(1, 1)

        def compute(ci, par):
            off0 = ci * _C

            def j_body(j, _):
                off = off0 + j * _LANES
                m = m_v[pl.ds(off, _LANES)]

                def r_body(r, carry):
                    mx1, mx2, ix = carry
                    v = ref_v[par, r, pl.ds(j * _LANES, _LANES)] * m
                    gt = v > mx1
                    mx2n = jnp.maximum(mx2, jnp.minimum(v, mx1))
                    ixn = jnp.where(gt, jnp.full((_LANES,), 0, jnp.int32) + r, ix)
                    mx1n = jnp.maximum(mx1, v)
                    return mx1n, mx2n, ixn

                neg = jnp.full((_LANES,), -jnp.inf, jnp.float32)
                mx1, mx2, ix = lax.fori_loop(
                    0, _R, r_body,
                    (neg, neg, jnp.zeros((_LANES,), jnp.int32)),
                    unroll=16)
                p_v[pl.ds(off, _LANES)] = mx1 * w0 + mx2 * w1
                i_v[pl.ds(off, _LANES)] = ix
                return 0

            lax.fori_loop(0, _C // _LANES, j_body, 0, unroll=2)

        def chunk_pair(cp, _):
            ci0 = cp * 2
            wait_in(0)
            compute(ci0, 0)

            @pl.when(ci0 + 2 < _NCHUNK)
            def _():
                start_in(ci0 + 2, 0)

            wait_in(1)
            compute(ci0 + 1, 1)

            @pl.when(ci0 + 3 < _NCHUNK)
            def _():
                start_in(ci0 + 3, 1)

            return 0

        lax.fori_loop(0, _NCHUNK // 2, chunk_pair, 0)

        pltpu.sync_copy(p_v, pooled_hbm.at[row, pl.ds(l_base, _LW)])
        pltpu.sync_copy(i_v, idx_hbm.at[row, pl.ds(l_base, _LW)])

    return body(mixed, ref, w1d)


def _tc_body(m_ref, r_ref, w_ref, p_ref, i_ref):
    blk = r_ref[...]
    m01 = m_ref[...]
    m2 = jnp.concatenate(
        [m01[0:1], m01[0:1], m01[1:2], m01[1:2]], axis=0)
    multi = blk * m2[:, None, :]
    mx1 = jnp.max(multi, axis=1, keepdims=True)
    iota = lax.broadcasted_iota(jnp.int32, (_ROWS_TC, _R, _CT), 1)
    ix = jnp.min(jnp.where(multi == mx1, iota, _R), axis=1, keepdims=True)
    masked = jnp.where(iota == ix, -jnp.inf, multi)
    mx2 = jnp.max(masked, axis=1, keepdims=True)
    w0 = w_ref[0, 0]
    w1 = w_ref[0, 1]
    p_ref[...] = (mx1 * w0 + mx2 * w1)[:, 0, :]
    i_ref[...] = ix[:, 0, :]


def _tc_call(mixed, ref, w1d):
    grid = (_L // _CT,)
    return pl.pallas_call(
        _tc_body,
        grid=grid,
        in_specs=[
            pl.BlockSpec((_B, _CT), lambda j: (0, j)),
            pl.BlockSpec((_ROWS_TC, _R, _CT), lambda j: (0, 0, j)),
            pl.BlockSpec((1, 128), lambda j: (0, 0)),
        ],
        out_specs=[
            pl.BlockSpec((_ROWS_TC, _CT), lambda j: (0, j)),
            pl.BlockSpec((_ROWS_TC, _CT), lambda j: (0, j)),
        ],
        out_shape=[
            jax.ShapeDtypeStruct((_ROWS_TC, _L), jnp.float32),
            jax.ShapeDtypeStruct((_ROWS_TC, _L), jnp.int32),
        ],
    )(mixed, ref, w1d)


def kernel(mixed_vcf, ref_panel, weights):
    ref = ref_panel.reshape(_BA, _R, _L)
    k = weights.shape[0]
    w1d = jnp.pad(weights.reshape(1, k), ((0, 0), (0, 128 - k)))

    p_tc, i_tc = _tc_call(mixed_vcf, ref, w1d)
    p_sc, i_sc = _sc_call(mixed_vcf, ref, w1d)

    pooled = jnp.concatenate(
        [p_tc.reshape(_ROWS_TC // _A, _A, _L),
         p_sc.reshape(_ROWS_SC // _A, _A, _L)], axis=0)
    idx = jnp.concatenate(
        [i_tc.reshape(_ROWS_TC // _A, _A, _L),
         i_sc.reshape(_ROWS_SC // _A, _A, _L)], axis=0)
    return pooled, idx


# speedup vs baseline: 1.9079x; 1.0059x over previous
import functools

import jax
import jax.numpy as jnp
from jax import lax
from jax.experimental import pallas as pl
from jax.experimental.pallas import tpu as pltpu
from jax.experimental.pallas import tpu_sc as plsc

_B, _A, _R, _L = 4, 2, 64, 65536
_BA = _B * _A
_NC, _NS, _LANES = 2, 16, 16
_NW = _NC * _NS

_ROWS_TC = 4
_ROWS_SC = _BA - _ROWS_TC
_WPR = _NW // _ROWS_SC
_LW = _L // _WPR
_C = 512
_NCHUNK = _LW // _C

_CT = 8192


def _sc_call(mixed, ref, w1d):
    mesh = plsc.VectorSubcoreMesh(core_axis_name="c", subcore_axis_name="s")

    @functools.partial(
        pl.kernel,
        mesh=mesh,
        out_type=[
            jax.ShapeDtypeStruct((_ROWS_SC // _A, _A, _L), jnp.float32),
            jax.ShapeDtypeStruct((_ROWS_SC // _A, _A, _L), jnp.int32),
        ],
        scratch_types=[
            pltpu.VMEM((2, _R, _C), jnp.float32),
            pltpu.VMEM((_LW,), jnp.float32),
            pltpu.VMEM((_LW,), jnp.float32),
            pltpu.VMEM((_LW,), jnp.int32),
            pltpu.VMEM((_LANES,), jnp.float32),
            pltpu.SemaphoreType.DMA,
            pltpu.SemaphoreType.DMA,
        ],
    )
    def body(mixed_hbm, ref_hbm, w_hbm, pooled_hbm, idx_hbm,
             ref_v, m_v, p_v, i_v, w_v, sem0, sem1):
        wid = lax.axis_index("s") * _NC + lax.axis_index("c")
        row = wid // _WPR
        ba = _ROWS_TC + row
        b = ba // _A
        l_base = (wid % _WPR) * _LW
        sems = (sem0, sem1)

        def start_in(ci, par):
            l0 = l_base + ci * _C
            pltpu.async_copy(
                ref_hbm.at[ba, :, pl.ds(l0, _C)], ref_v.at[par], sems[par])

        def wait_in(par):
            pltpu.make_async_copy(
                ref_hbm.at[ba, :, pl.ds(l_base, _C)], ref_v.at[par],
                sems[par]).wait()

        pltpu.sync_copy(w_hbm.at[0, pl.ds(0, _LANES)], w_v)
        pltpu.sync_copy(mixed_hbm.at[b, pl.ds(l_base, _LW)], m_v)
        wvec = w_v[...]
        w0 = wvec[0]
        w1 = wvec[1]

        start_in(0, 0)
        start_in(1, 1)

        def compute(ci, par):
            off0 = ci * _C

            def j_body(j, _):
                off = off0 + j * _LANES
                m = m_v[pl.ds(off, _LANES)]

                def r_body(r, carry):
                    mx1, mx2, ix = carry
                    v = ref_v[par, r, pl.ds(j * _LANES, _LANES)] * m
                    gt = v > mx1
                    mx2n = jnp.maximum(mx2, jnp.minimum(v, mx1))
                    ixn = jnp.where(gt, jnp.full((_LANES,), 0, jnp.int32) + r, ix)
                    mx1n = jnp.maximum(mx1, v)
                    return mx1n, mx2n, ixn

                neg = jnp.full((_LANES,), -jnp.inf, jnp.float32)
                mx1, mx2, ix = lax.fori_loop(
                    0, _R, r_body,
                    (neg, neg, jnp.zeros((_LANES,), jnp.int32)),
                    unroll=16)
                p_v[pl.ds(off, _LANES)] = mx1 * w0 + mx2 * w1
                i_v[pl.ds(off, _LANES)] = ix
                return 0

            lax.fori_loop(0, _C // _LANES, j_body, 0, unroll=2)

        def chunk_pair(cp, _):
            ci0 = cp * 2
            wait_in(0)
            compute(ci0, 0)

            @pl.when(ci0 + 2 < _NCHUNK)
            def _():
                start_in(ci0 + 2, 0)

            wait_in(1)
            compute(ci0 + 1, 1)

            @pl.when(ci0 + 3 < _NCHUNK)
            def _():
                start_in(ci0 + 3, 1)

            return 0

        lax.fori_loop(0, _NCHUNK // 2, chunk_pair, 0)

        pltpu.sync_copy(p_v, pooled_hbm.at[row // _A, row % _A, pl.ds(l_base, _LW)])
        pltpu.sync_copy(i_v, idx_hbm.at[row // _A, row % _A, pl.ds(l_base, _LW)])

    return body(mixed, ref, w1d)


def _tc_body(m_ref, r_ref, w_ref, p_ref, i_ref):
    blk = r_ref[...]
    m01 = m_ref[...]
    m2 = jnp.concatenate(
        [m01[0:1], m01[0:1], m01[1:2], m01[1:2]], axis=0)
    multi = blk * m2[:, None, :]
    mx1 = jnp.max(multi, axis=1, keepdims=True)
    iota = lax.broadcasted_iota(jnp.int32, (_ROWS_TC, _R, _CT), 1)
    ix = jnp.min(jnp.where(multi == mx1, iota, _R), axis=1, keepdims=True)
    masked = jnp.where(iota == ix, -jnp.inf, multi)
    mx2 = jnp.max(masked, axis=1, keepdims=True)
    w0 = w_ref[0, 0]
    w1 = w_ref[0, 1]
    p_ref[...] = (mx1 * w0 + mx2 * w1)[:, 0, :].reshape(
        _ROWS_TC // _A, _A, _CT)
    i_ref[...] = ix[:, 0, :].reshape(_ROWS_TC // _A, _A, _CT)


def _tc_call(mixed, ref, w1d):
    grid = (_L // _CT,)
    return pl.pallas_call(
        _tc_body,
        grid=grid,
        in_specs=[
            pl.BlockSpec((_B, _CT), lambda j: (0, j)),
            pl.BlockSpec((_ROWS_TC, _R, _CT), lambda j: (0, 0, j)),
            pl.BlockSpec((1, 128), lambda j: (0, 0)),
        ],
        out_specs=[
            pl.BlockSpec((_ROWS_TC // _A, _A, _CT), lambda j: (0, 0, j)),
            pl.BlockSpec((_ROWS_TC // _A, _A, _CT), lambda j: (0, 0, j)),
        ],
        out_shape=[
            jax.ShapeDtypeStruct((_ROWS_TC // _A, _A, _L), jnp.float32),
            jax.ShapeDtypeStruct((_ROWS_TC // _A, _A, _L), jnp.int32),
        ],
    )(mixed, ref, w1d)


def kernel(mixed_vcf, ref_panel, weights):
    ref = ref_panel.reshape(_BA, _R, _L)
    k = weights.shape[0]
    w1d = jnp.pad(weights.reshape(1, k), ((0, 0), (0, 128 - k)))

    p_tc, i_tc = _tc_call(mixed_vcf, ref, w1d)
    p_sc, i_sc = _sc_call(mixed_vcf, ref, w1d)

    pooled = jnp.concatenate([p_tc, p_sc], axis=0)
    idx = jnp.concatenate([i_tc, i_sc], axis=0)
    return pooled, idx


# speedup vs baseline: 1.9207x; 1.0067x over previous
import functools

import jax
import jax.numpy as jnp
from jax import lax
from jax.experimental import pallas as pl
from jax.experimental.pallas import tpu as pltpu
from jax.experimental.pallas import tpu_sc as plsc

_B, _A, _R, _L = 4, 2, 64, 65536
_BA = _B * _A
_NC, _NS, _LANES = 2, 16, 16
_NW = _NC * _NS

_ROWS_TC = 4
_ROWS_SC = _BA - _ROWS_TC
_XCOL = 4096
_WPR = _NW // _ROWS_SC
_LW = (_L - _XCOL) // _WPR
_C = 512
_NCHUNK = _LW // _C
_NBUF = 3

_CT = 8192


def _sc_call(mixed, ref, w1d):
    mesh = plsc.VectorSubcoreMesh(core_axis_name="c", subcore_axis_name="s")

    @functools.partial(
        pl.kernel,
        mesh=mesh,
        out_type=[
            jax.ShapeDtypeStruct((_ROWS_SC // _A, _A, _L - _XCOL), jnp.float32),
            jax.ShapeDtypeStruct((_ROWS_SC // _A, _A, _L - _XCOL), jnp.int32),
        ],
        scratch_types=[
            pltpu.VMEM((_NBUF, _R, _C), jnp.float32),
            pltpu.VMEM((_LW,), jnp.float32),
            pltpu.VMEM((_LW,), jnp.float32),
            pltpu.VMEM((_LW,), jnp.int32),
            pltpu.VMEM((_LANES,), jnp.float32),
            pltpu.SemaphoreType.DMA,
            pltpu.SemaphoreType.DMA,
            pltpu.SemaphoreType.DMA,
        ],
    )
    def body(mixed_hbm, ref_hbm, w_hbm, pooled_hbm, idx_hbm,
             ref_v, m_v, p_v, i_v, w_v, sem0, sem1, sem2):
        wid = lax.axis_index("s") * _NC + lax.axis_index("c")
        row = wid // _WPR
        ba = _ROWS_TC + row
        b = ba // _A
        span0 = (wid % _WPR) * _LW
        l_base = _XCOL + span0
        sems = (sem0, sem1, sem2)

        def start_in(ci, par):
            l0 = l_base + ci * _C
            pltpu.async_copy(
                ref_hbm.at[ba, :, pl.ds(l0, _C)], ref_v.at[par], sems[par])

        def wait_in(par):
            pltpu.make_async_copy(
                ref_hbm.at[ba, :, pl.ds(l_base, _C)], ref_v.at[par],
                sems[par]).wait()

        pltpu.sync_copy(w_hbm.at[0, pl.ds(0, _LANES)], w_v)
        pltpu.sync_copy(mixed_hbm.at[b, pl.ds(l_base, _LW)], m_v)
        wvec = w_v[...]
        w0 = wvec[0]
        w1 = wvec[1]

        for par in range(_NBUF):
            start_in(par, par)

        def compute(ci, par):
            off0 = ci * _C

            def j_body(j, _):
                off = off0 + j * _LANES
                m = m_v[pl.ds(off, _LANES)]

                def r_body(r, carry):
                    mx1, mx2, ix = carry
                    v = ref_v[par, r, pl.ds(j * _LANES, _LANES)] * m
                    gt = v > mx1
                    mx2n = jnp.maximum(mx2, jnp.minimum(v, mx1))
                    ixn = jnp.where(gt, jnp.full((_LANES,), 0, jnp.int32) + r, ix)
                    mx1n = jnp.maximum(mx1, v)
                    return mx1n, mx2n, ixn

                neg = jnp.full((_LANES,), -jnp.inf, jnp.float32)
                mx1, mx2, ix = lax.fori_loop(
                    0, _R, r_body,
                    (neg, neg, jnp.zeros((_LANES,), jnp.int32)),
                    unroll=16)
                p_v[pl.ds(off, _LANES)] = mx1 * w0 + mx2 * w1
                i_v[pl.ds(off, _LANES)] = ix
                return 0

            lax.fori_loop(0, _C // _LANES, j_body, 0, unroll=2)

        def chunk_group(cg, _):
            ci0 = cg * _NBUF
            for par in range(_NBUF):
                ci = ci0 + par
                wait_in(par)
                compute(ci, par)

                @pl.when(ci + _NBUF < _NCHUNK)
                def _():
                    start_in(ci + _NBUF, par)

            return 0

        lax.fori_loop(0, _NCHUNK // _NBUF, chunk_group, 0)

        pltpu.sync_copy(p_v, pooled_hbm.at[row // _A, row % _A,
                                           pl.ds(span0, _LW)])
        pltpu.sync_copy(i_v, idx_hbm.at[row // _A, row % _A,
                                        pl.ds(span0, _LW)])

    return body(mixed, ref, w1d)


def _make_tc_body(row_block, ct):
    b0 = row_block * (_ROWS_TC // _A)

    def tc_body(m_ref, r_ref, w_ref, p_ref, i_ref):
        blk = r_ref[...]
        m01 = m_ref[...]
        m2 = jnp.concatenate(
            [m01[b0:b0 + 1], m01[b0:b0 + 1],
             m01[b0 + 1:b0 + 2], m01[b0 + 1:b0 + 2]], axis=0)
        multi = blk * m2[:, None, :]
        mx1 = jnp.max(multi, axis=1, keepdims=True)
        iota = lax.broadcasted_iota(jnp.int32, (4, _R, ct), 1)
        ix = jnp.min(jnp.where(multi == mx1, iota, _R), axis=1, keepdims=True)
        masked = jnp.where(iota == ix, -jnp.inf, multi)
        mx2 = jnp.max(masked, axis=1, keepdims=True)
        w0 = w_ref[0, 0]
        w1 = w_ref[0, 1]
        p_ref[...] = (mx1 * w0 + mx2 * w1)[:, 0, :].reshape(2, _A, ct)
        i_ref[...] = ix[:, 0, :].reshape(2, _A, ct)

    return tc_body


def _tc_call_main(mixed, ref, w1d):
    grid = (_L // _CT,)
    return pl.pallas_call(
        _make_tc_body(0, _CT),
        grid=grid,
        in_specs=[
            pl.BlockSpec((_B, _CT), lambda j: (0, j)),
            pl.BlockSpec((_ROWS_TC, _R, _CT), lambda j: (0, 0, j)),
            pl.BlockSpec((1, 128), lambda j: (0, 0)),
        ],
        out_specs=[
            pl.BlockSpec((2, _A, _CT), lambda j: (0, 0, j)),
            pl.BlockSpec((2, _A, _CT), lambda j: (0, 0, j)),
        ],
        out_shape=[
            jax.ShapeDtypeStruct((2, _A, _L), jnp.float32),
            jax.ShapeDtypeStruct((2, _A, _L), jnp.int32),
        ],
    )(mixed, ref, w1d)


def _tc_call_edge(mixed, ref, w1d):
    return pl.pallas_call(
        _make_tc_body(1, _XCOL),
        grid=(1,),
        in_specs=[
            pl.BlockSpec((_B, _XCOL), lambda j: (0, 0)),
            pl.BlockSpec((_ROWS_SC, _R, _XCOL), lambda j: (1, 0, 0)),
            pl.BlockSpec((1, 128), lambda j: (0, 0)),
        ],
        out_specs=[
            pl.BlockSpec((2, _A, _XCOL), lambda j: (0, 0, 0)),
            pl.BlockSpec((2, _A, _XCOL), lambda j: (0, 0, 0)),
        ],
        out_shape=[
            jax.ShapeDtypeStruct((2, _A, _XCOL), jnp.float32),
            jax.ShapeDtypeStruct((2, _A, _XCOL), jnp.int32),
        ],
    )(mixed, ref, w1d)


def kernel(mixed_vcf, ref_panel, weights):
    ref = ref_panel.reshape(_BA, _R, _L)
    k = weights.shape[0]
    w1d = jnp.pad(weights.reshape(1, k), ((0, 0), (0, 128 - k)))

    p_tc, i_tc = _tc_call_main(mixed_vcf, ref, w1d)
    p_te, i_te = _tc_call_edge(mixed_vcf, ref, w1d)
    p_sc, i_sc = _sc_call(mixed_vcf, ref, w1d)

    p_lo = jnp.concatenate([p_te, p_sc], axis=2)
    i_lo = jnp.concatenate([i_te, i_sc], axis=2)
    pooled = jnp.concatenate([p_tc, p_lo], axis=0)
    idx = jnp.concatenate([i_tc, i_lo], axis=0)
    return pooled, idx


# speedup vs baseline: 1.9695x; 1.0254x over previous
import functools

import jax
import jax.numpy as jnp
from jax import lax
from jax.experimental import pallas as pl
from jax.experimental.pallas import tpu as pltpu
from jax.experimental.pallas import tpu_sc as plsc

_B, _A, _R, _L = 4, 2, 64, 65536
_BA = _B * _A
_NC, _NS, _LANES = 2, 16, 16
_NW = _NC * _NS

_ROWS_TC = 4
_ROWS_SC = _BA - _ROWS_TC
_XCOL = 8192
_WPR = _NW // _ROWS_SC
_LW = (_L - _XCOL) // _WPR
_C = 512
_NCHUNK = _LW // _C
_NBUF = 2

_CT = 8192


def _sc_call(mixed, ref, w1d):
    mesh = plsc.VectorSubcoreMesh(core_axis_name="c", subcore_axis_name="s")

    @functools.partial(
        pl.kernel,
        mesh=mesh,
        out_type=[
            jax.ShapeDtypeStruct((_ROWS_SC // _A, _A, _L - _XCOL), jnp.float32),
            jax.ShapeDtypeStruct((_ROWS_SC // _A, _A, _L - _XCOL), jnp.int32),
        ],
        scratch_types=[
            pltpu.VMEM((_NBUF, _R, _C), jnp.float32),
            pltpu.VMEM((_LW,), jnp.float32),
            pltpu.VMEM((_LW,), jnp.float32),
            pltpu.VMEM((_LW,), jnp.int32),
            pltpu.VMEM((_LANES,), jnp.float32),
            pltpu.SemaphoreType.DMA,
            pltpu.SemaphoreType.DMA,
        ],
    )
    def body(mixed_hbm, ref_hbm, w_hbm, pooled_hbm, idx_hbm,
             ref_v, m_v, p_v, i_v, w_v, sem0, sem1):
        wid = lax.axis_index("s") * _NC + lax.axis_index("c")
        row = wid // _WPR
        ba = _ROWS_TC + row
        b = ba // _A
        span0 = (wid % _WPR) * _LW
        l_base = _XCOL + span0
        sems = (sem0, sem1)

        def start_in(ci, par):
            l0 = l_base + ci * _C
            pltpu.async_copy(
                ref_hbm.at[ba, :, pl.ds(l0, _C)], ref_v.at[par], sems[par])

        def wait_in(par):
            pltpu.make_async_copy(
                ref_hbm.at[ba, :, pl.ds(l_base, _C)], ref_v.at[par],
                sems[par]).wait()

        pltpu.sync_copy(w_hbm.at[0, pl.ds(0, _LANES)], w_v)
        pltpu.sync_copy(mixed_hbm.at[b, pl.ds(l_base, _LW)], m_v)
        wvec = w_v[...]
        w0 = wvec[0]
        w1 = wvec[1]

        for par in range(_NBUF):
            start_in(par, par)

        def compute(ci, par):
            off0 = ci * _C

            def j_body(j, _):
                off = off0 + j * _LANES
                m = m_v[pl.ds(off, _LANES)]

                def r_body(r, carry):
                    mx1, mx2, ix = carry
                    v = ref_v[par, r, pl.ds(j * _LANES, _LANES)] * m
                    gt = v > mx1
                    mx2n = jnp.maximum(mx2, jnp.minimum(v, mx1))
                    ixn = jnp.where(gt, jnp.full((_LANES,), 0, jnp.int32) + r, ix)
                    mx1n = jnp.maximum(mx1, v)
                    return mx1n, mx2n, ixn

                neg = jnp.full((_LANES,), -jnp.inf, jnp.float32)
                mx1, mx2, ix = lax.fori_loop(
                    0, _R, r_body,
                    (neg, neg, jnp.zeros((_LANES,), jnp.int32)),
                    unroll=16)
                p_v[pl.ds(off, _LANES)] = mx1 * w0 + mx2 * w1
                i_v[pl.ds(off, _LANES)] = ix
                return 0

            lax.fori_loop(0, _C // _LANES, j_body, 0, unroll=2)

        def chunk_group(cg, _):
            ci0 = cg * _NBUF
            for par in range(_NBUF):
                ci = ci0 + par
                wait_in(par)
                compute(ci, par)

                @pl.when(ci + _NBUF < _NCHUNK)
                def _():
                    start_in(ci + _NBUF, par)

            return 0

        lax.fori_loop(0, _NCHUNK // _NBUF, chunk_group, 0)

        pltpu.sync_copy(p_v, pooled_hbm.at[row // _A, row % _A,
                                           pl.ds(span0, _LW)])
        pltpu.sync_copy(i_v, idx_hbm.at[row // _A, row % _A,
                                        pl.ds(span0, _LW)])

    return body(mixed, ref, w1d)


def _make_tc_body(row_block, ct):
    b0 = row_block * (_ROWS_TC // _A)

    def tc_body(m_ref, r_ref, w_ref, p_ref, i_ref):
        blk = r_ref[...]
        m01 = m_ref[...]
        m2 = jnp.concatenate(
            [m01[b0:b0 + 1], m01[b0:b0 + 1],
             m01[b0 + 1:b0 + 2], m01[b0 + 1:b0 + 2]], axis=0)
        multi = blk * m2[:, None, :]
        mx1 = jnp.max(multi, axis=1, keepdims=True)
        iota = lax.broadcasted_iota(jnp.int32, (4, _R, ct), 1)
        ix = jnp.min(jnp.where(multi == mx1, iota, _R), axis=1, keepdims=True)
        masked = jnp.where(iota == ix, -jnp.inf, multi)
        mx2 = jnp.max(masked, axis=1, keepdims=True)
        w0 = w_ref[0, 0]
        w1 = w_ref[1, 0]
        p_ref[...] = (mx1 * w0 + mx2 * w1)[:, 0, :].reshape(2, _A, ct)
        i_ref[...] = ix[:, 0, :].reshape(2, _A, ct)

    return tc_body


def _tc_call_main(mixed, ref, w1d):
    grid = (_L // _CT,)
    return pl.pallas_call(
        _make_tc_body(0, _CT),
        grid=grid,
        in_specs=[
            pl.BlockSpec((_B, _CT), lambda j: (0, j)),
            pl.BlockSpec((_ROWS_TC, _R, _CT), lambda j: (0, 0, j)),
            pl.BlockSpec((2, 1), lambda j: (0, 0)),
        ],
        out_specs=[
            pl.BlockSpec((2, _A, _CT), lambda j: (0, 0, j)),
            pl.BlockSpec((2, _A, _CT), lambda j: (0, 0, j)),
        ],
        out_shape=[
            jax.ShapeDtypeStruct((2, _A, _L), jnp.float32),
            jax.ShapeDtypeStruct((2, _A, _L), jnp.int32),
        ],
    )(mixed, ref, w1d)


def _tc_call_edge(mixed, ref, w1d):
    return pl.pallas_call(
        _make_tc_body(1, _XCOL),
        grid=(1,),
        in_specs=[
            pl.BlockSpec((_B, _XCOL), lambda j: (0, 0)),
            pl.BlockSpec((_ROWS_SC, _R, _XCOL), lambda j: (1, 0, 0)),
            pl.BlockSpec((2, 1), lambda j: (0, 0)),
        ],
        out_specs=[
            pl.BlockSpec((2, _A, _XCOL), lambda j: (0, 0, 0)),
            pl.BlockSpec((2, _A, _XCOL), lambda j: (0, 0, 0)),
        ],
        out_shape=[
            jax.ShapeDtypeStruct((2, _A, _XCOL), jnp.float32),
            jax.ShapeDtypeStruct((2, _A, _XCOL), jnp.int32),
        ],
    )(mixed, ref, w1d)


def kernel(mixed_vcf, ref_panel, weights):
    ref = ref_panel.reshape(_BA, _R, _L)
    k = weights.shape[0]
    w1d = jnp.pad(weights.reshape(1, k), ((0, 0), (0, 128 - k)))

    p_tc, i_tc = _tc_call_main(mixed_vcf, ref, weights)
    p_te, i_te = _tc_call_edge(mixed_vcf, ref, weights)
    p_sc, i_sc = _sc_call(mixed_vcf, ref, w1d)

    p_lo = jnp.concatenate([p_te, p_sc], axis=2)
    i_lo = jnp.concatenate([i_te, i_sc], axis=2)
    pooled = jnp.concatenate([p_tc, p_lo], axis=0)
    idx = jnp.concatenate([i_tc, i_lo], axis=0)
    return pooled, idx
